# baseline (speedup 1.0000x reference)
import functools

import jax
import jax.numpy as jnp
from jax import lax
from jax.experimental import pallas as pl
from jax.experimental.pallas import tpu as pltpu
from jax.experimental.pallas import tpu_sc as plsc

F32 = jnp.float32
I32 = jnp.int32

N_NODES = 10000
N_EDGES = 320000
D = 128
N_GRAPHS = 256

NP = 10240
NB = NP // 128
R = 2560
RPW = R // 32
NEPW = RPW * 128
R2 = R * 2
NPAD_E = R * 128 - N_EDGES
TILE_N = NP // 16

_mesh = lambda: plsc.VectorSubcoreMesh(core_axis_name="c", subcore_axis_name="s")


def _lane_bcast(v16, i):
    idx = jnp.full((16, 1), i, dtype=I32)
    dn = lax.GatherDimensionNumbers(
        offset_dims=(), collapsed_slice_dims=(0,), start_index_map=(0,))
    return lax.gather(v16, idx, dn, (1,),
                      mode=lax.GatherScatterMode.PROMISE_IN_BOUNDS)


@functools.partial(
    pl.kernel,
    out_type=(jax.ShapeDtypeStruct((R, 128), F32),
              jax.ShapeDtypeStruct((2 * NP,), F32)),
    mesh=_mesh(),
    scratch_types=[
        pltpu.VMEM((RPW, 128), I32),
        pltpu.VMEM((RPW, 128), I32),
        pltpu.VMEM((NEPW,), I32),
        pltpu.VMEM((NEPW,), I32),
        pltpu.VMEM((NEPW,), F32),
        pltpu.VMEM((NEPW,), F32),
        pltpu.VMEM((RPW, 128), F32),
        pltpu.VMEM((TILE_N,), F32),
        pltpu.VMEM_SHARED((NP,), F32),
        pltpu.SemaphoreType.DMA,
        pltpu.SemaphoreType.DMA,
    ],
)
def _sc_soft(hA_h, src_h, dst_h, ex_h, den_h,
             srcs, dsts, idxs, idxd, asf, adf, exs, zb, den_sh, sem1, sem2):
    c = lax.axis_index("c")
    s = lax.axis_index("s")
    w = s * 2 + c

    def _zb(i, carry):
        zb[pl.ds(i * 16, 16)] = jnp.zeros((16,), F32)
        return carry
    lax.fori_loop(0, TILE_N // 16, _zb, 0)
    pltpu.sync_copy(zb, den_sh.at[pl.ds(s * TILE_N, TILE_N)])

    base = w * RPW
    pltpu.sync_copy(src_h.at[pl.ds(base, RPW)], srcs)
    pltpu.sync_copy(dst_h.at[pl.ds(base, RPW)], dsts)

    def _idx(r, carry):
        for j in range(8):
            sl = pl.ds(j * 16, 16)
            fl = pl.ds(r * 128 + j * 16, 16)
            idxs[fl] = srcs[r, sl] * 128
            idxd[fl] = dsts[r, sl] * 128 + 1
        return carry
    lax.fori_loop(0, RPW, _idx, 0)

    cp1 = pltpu.async_copy(hA_h.at[idxs], asf, sem1)
    cp2 = pltpu.async_copy(hA_h.at[idxd], adf, sem2)
    cp1.wait()
    cp2.wait()

    def _ex(r, carry):
        for j in range(8):
            sl = pl.ds(j * 16, 16)
            fl = pl.ds(r * 128 + j * 16, 16)
            e = asf[fl] + adf[fl]
            e = jnp.where(e >= 0.0, e, e * 0.2)
            exs[r, sl] = jnp.exp(e)
        return carry
    lax.fori_loop(0, RPW, _ex, 0)

    cp3 = pltpu.async_copy(exs, ex_h.at[pl.ds(base, RPW)], sem1)
    plsc.subcore_barrier()

    def _fire(r, carry):
        pltpu.async_copy(exs.at[r], den_sh.at[dsts.at[r]], sem2, add=True)
        return carry
    lax.fori_loop(0, RPW, _fire, 0)

    def _drain(r, carry):
        pltpu.make_async_copy(exs.at[0], den_sh.at[pl.ds(0, 128)], sem2).wait()
        return carry
    lax.fori_loop(0, RPW, _drain, 0)
    cp3.wait()

    plsc.subcore_barrier()
    pltpu.sync_copy(den_sh.at[pl.ds(s * TILE_N, TILE_N)],
                    den_h.at[pl.ds(c * NP + s * TILE_N, TILE_N)])


SPC = 40


@functools.partial(
    pl.kernel,
    out_type=jax.ShapeDtypeStruct((2 * NP, 128), F32),
    mesh=_mesh(),
    scratch_types=[
        pltpu.VMEM((SPC, 64), I32),
        pltpu.VMEM((SPC, 64), I32),
        pltpu.VMEM((SPC, 128), F32),
        pltpu.VMEM((64, 128), F32),
        pltpu.VMEM((64, 128), F32),
        pltpu.VMEM((64, 128), F32),
        pltpu.VMEM((64, 128), F32),
        pltpu.VMEM_SHARED((NP, 128), F32),
        pltpu.SemaphoreType.DMA,
        pltpu.SemaphoreType.DMA,
        pltpu.SemaphoreType.DMA,
        pltpu.SemaphoreType.DMA,
        pltpu.SemaphoreType.DMA,
        pltpu.SemaphoreType.DMA,
        pltpu.SemaphoreType.DMA,
        pltpu.SemaphoreType.DMA,
    ],
)
def _sc_agg(h_h, src_h, dst_h, ex_h, out_h,
            srcs, dsts, exs, rows0, rows1, rows2, rows3, acc_sh,
            g0, g1, g2, g3, s0, s1, s2, s3):
    c = lax.axis_index("c")
    s = lax.axis_index("s")
    w = s * 2 + c
    bufs = (rows0, rows1, rows2, rows3)
    gsems = (g0, g1, g2, g3)
    ssems = (s0, s1, s2, s3)

    def _zr(i, carry):
        for j in range(8):
            rows0[i, pl.ds(j * 16, 16)] = jnp.zeros((16,), F32)
        return carry
    lax.fori_loop(0, 64, _zr, 0)

    def _za(k, carry):
        pltpu.sync_copy(rows0, acc_sh.at[pl.ds(s * TILE_N + k * 64, 64)])
        return carry
    lax.fori_loop(0, TILE_N // 64, _za, 0)
    plsc.subcore_barrier()

    def _scale(buf, exrow, colbase):
        def _grp(g, carry):
            a16 = exs[exrow, pl.ds(colbase + g * 16, 16)]
            for i2 in range(16):
                b = _lane_bcast(a16, i2)
                row = g * 16 + i2
                for j in range(8):
                    sl = pl.ds(j * 16, 16)
                    buf[row, sl] = buf[row, sl] * b
            return carry
        lax.fori_loop(0, 4, _grp, 0)

    def _sub(base, exoff):
        pltpu.sync_copy(src_h.at[pl.ds(base, SPC)], srcs)
        pltpu.sync_copy(dst_h.at[pl.ds(base, SPC)], dsts)

        def _step(cl, i, t, first, tail):
            pltpu.make_async_copy(h_h.at[pl.ds(0, 64)], bufs[i],
                                  gsems[i]).wait()
            _scale(bufs[i], exoff + 2 * t + i // 2, (i % 2) * 64)
            prev = (i + 3) % 4
            if not first:
                pltpu.make_async_copy(bufs[prev], acc_sh.at[pl.ds(0, 64)],
                                      ssems[prev]).wait()
            if not tail:
                pltpu.async_copy(h_h.at[srcs.at[cl + 3]], bufs[prev],
                                 gsems[prev])
            pltpu.async_copy(bufs[i], acc_sh.at[dsts.at[cl]], ssems[i],
                             add=True)

        for i in range(3):
            pltpu.async_copy(h_h.at[srcs.at[i]], bufs[i], gsems[i])
        for i in range(4):
            _step(i, i, 0, first=(i == 0), tail=False)

        def _quad(t, carry):
            for i in range(4):
                _step(4 * t + i, i, t, first=False, tail=False)
            return carry
        lax.fori_loop(1, SPC // 4 - 1, _quad, 0)

        t_last = SPC // 4 - 1
        for i in range(4):
            cl = 4 * t_last + i
            _step(cl, i, t_last, first=False, tail=(cl + 3 >= SPC))
        pltpu.make_async_copy(bufs[3], acc_sh.at[pl.ds(0, 64)],
                              ssems[3]).wait()

    for half in range(2):
        hbase = w * 160 + half * 80
        pltpu.sync_copy(ex_h.at[pl.ds(w * 80 + half * 40, SPC)], exs)
        for sub in range(2):
            _sub(hbase + sub * SPC, sub * 20)

    plsc.subcore_barrier()
    pltpu.sync_copy(acc_sh.at[pl.ds(s * TILE_N, TILE_N)],
                    out_h.at[pl.ds(c * NP + s * TILE_N, TILE_N)])


def _tc_dense_body(x_ref, w_ref, a_ref, h_ref, ha_ref):
    k = pl.program_id(0)
    gid = k * 128 + lax.broadcasted_iota(I32, (128, 1), 0)
    xv = jnp.where(gid < N_NODES, x_ref[...], 0.0)
    h = jnp.dot(xv, w_ref[...], preferred_element_type=F32)
    h_ref[...] = h
    ha_ref[...] = jnp.dot(h, a_ref[...], preferred_element_type=F32)


def _tc_dense(x, W, A):
    nxb = (N_NODES + 127) // 128 - 1
    return pl.pallas_call(
        _tc_dense_body,
        grid=(NB,),
        in_specs=[
            pl.BlockSpec((128, 128), lambda k: (jnp.minimum(k, nxb), 0)),
            pl.BlockSpec((128, 128), lambda k: (0, 0)),
            pl.BlockSpec((128, 128), lambda k: (0, 0)),
        ],
        out_specs=[
            pl.BlockSpec((128, 128), lambda k: (k, 0)),
            pl.BlockSpec((128, 128), lambda k: (k, 0)),
        ],
        out_shape=[
            jax.ShapeDtypeStruct((NP, 128), F32),
            jax.ShapeDtypeStruct((NP, 128), F32),
        ],
    )(x, W, A)


def _tc_mid_body(o0_ref, o1_ref, d0_ref, d1_ref, b_ref, w_ref, a_ref,
                 h_ref, ha_ref):
    den = d0_ref[...] + d1_ref[...] + 1e-16
    h1 = jnp.maximum((o0_ref[...] + o1_ref[...]) / den + b_ref[...], 0.0)
    h2 = jnp.dot(h1, w_ref[...], preferred_element_type=F32)
    h_ref[...] = h2
    ha_ref[...] = jnp.dot(h2, a_ref[...], preferred_element_type=F32)


def _tc_mid(out1, den, b, W, A):
    return pl.pallas_call(
        _tc_mid_body,
        grid=(NB,),
        in_specs=[
            pl.BlockSpec((128, 128), lambda k: (k, 0)),
            pl.BlockSpec((128, 128), lambda k: (k + NB, 0)),
            pl.BlockSpec((128, 1), lambda k: (k, 0)),
            pl.BlockSpec((128, 1), lambda k: (k + NB, 0)),
            pl.BlockSpec((1, 128), lambda k: (0, 0)),
            pl.BlockSpec((128, 128), lambda k: (0, 0)),
            pl.BlockSpec((128, 128), lambda k: (0, 0)),
        ],
        out_specs=[
            pl.BlockSpec((128, 128), lambda k: (k, 0)),
            pl.BlockSpec((128, 128), lambda k: (k, 0)),
        ],
        out_shape=[
            jax.ShapeDtypeStruct((NP, 128), F32),
            jax.ShapeDtypeStruct((NP, 128), F32),
        ],
    )(out1, out1, den, den, b, W, A)


def _tc_fin_body(o0_ref, o1_ref, d0_ref, d1_ref, b_ref, wm1_ref, bm1_ref,
                 wm2_ref, bm2_ref, bt_ref, out_ref, acc):
    k = pl.program_id(0)
    den = d0_ref[...] + d1_ref[...] + 1e-16
    h2 = jnp.maximum((o0_ref[...] + o1_ref[...]) / den + b_ref[...], 0.0)
    t = jnp.maximum(
        jnp.dot(h2, wm1_ref[...], preferred_element_type=F32) + bm1_ref[...],
        0.0)
    z = jnp.dot(t, wm2_ref[...], preferred_element_type=F32) + bm2_ref[...]
    zext = jnp.concatenate(
        [z, jnp.ones((128, 1), F32), jnp.zeros((128, 127), F32)], axis=1)
    btv = bt_ref[0, 0, :]
    gi = lax.broadcasted_iota(I32, (N_GRAPHS, 128), 0)
    oh = (gi == btv[None, :]).astype(F32)
    contrib = jnp.dot(oh, zext, preferred_element_type=F32)

    @pl.when(k == 0)
    def _():
        acc[...] = jnp.zeros_like(acc)

    acc[...] += contrib

    @pl.when(k == NB - 1)
    def _():
        a = acc[...]
        out_ref[...] = a[:, :128] / jnp.maximum(a[:, 128:129], 1.0)


def _tc_fin(out2, den, b, Wm1, bm1, Wm2, bm2, batch3):
    return pl.pallas_call(
        _tc_fin_body,
        grid=(NB,),
        in_specs=[
            pl.BlockSpec((128, 128), lambda k: (k, 0)),
            pl.BlockSpec((128, 128), lambda k: (k + NB, 0)),
            pl.BlockSpec((128, 1), lambda k: (k, 0)),
            pl.BlockSpec((128, 1), lambda k: (k + NB, 0)),
            pl.BlockSpec((1, 128), lambda k: (0, 0)),
            pl.BlockSpec((128, 128), lambda k: (0, 0)),
            pl.BlockSpec((1, 128), lambda k: (0, 0)),
            pl.BlockSpec((128, 128), lambda k: (0, 0)),
            pl.BlockSpec((1, 128), lambda k: (0, 0)),
            pl.BlockSpec((1, 1, 128), lambda k: (k, 0, 0)),
        ],
        out_specs=pl.BlockSpec((N_GRAPHS, 128), lambda k: (0, 0)),
        out_shape=jax.ShapeDtypeStruct((N_GRAPHS, 128), F32),
        scratch_shapes=[pltpu.VMEM((N_GRAPHS, 256), F32)],
    )(out2, out2, den, den, b, Wm1, bm1, Wm2, bm2, batch3)


def kernel(x, edge_index, batch, W1, a1_src, a1_dst, b1,
           W2, a2_src, a2_dst, b2, Wm1, bm1, Wm2, bm2):
    src = edge_index[0].astype(I32)
    dst = edge_index[1].astype(I32)
    pad_idx = N_NODES + (jnp.arange(NPAD_E, dtype=I32) % (NP - N_NODES))
    src2 = jnp.concatenate([src, pad_idx]).reshape(R, 128)
    dst2 = jnp.concatenate([dst, pad_idx]).reshape(R, 128)
    src2b = src2.reshape(R2, 64)
    dst2b = dst2.reshape(R2, 64)
    A1 = jnp.concatenate(
        [a1_src[:, None], a1_dst[:, None], jnp.zeros((D, 126), F32)], axis=1)
    A2 = jnp.concatenate(
        [a2_src[:, None], a2_dst[:, None], jnp.zeros((D, 126), F32)], axis=1)
    batch3 = jnp.concatenate(
        [batch.astype(I32),
         jnp.full((NP - N_NODES,), N_GRAPHS, I32)]).reshape(NB, 1, 128)

    h1p, hA1 = _tc_dense(x, W1, A1)
    ex1, den1 = _sc_soft(hA1.reshape(NP * 128), src2, dst2)
    out1 = _sc_agg(h1p, src2b, dst2b, ex1)

    h2p, hA2 = _tc_mid(out1, den1.reshape(2 * NP, 1), b1.reshape(1, D),
                       W2, A2)
    ex2, den2 = _sc_soft(hA2.reshape(NP * 128), src2, dst2)
    out2 = _sc_agg(h2p, src2b, dst2b, ex2)

    return _tc_fin(out2, den2.reshape(2 * NP, 1), b2.reshape(1, D),
                   Wm1, bm1.reshape(1, D), Wm2, bm2.reshape(1, D), batch3)
